# Initial kernel scaffold; baseline (speedup 1.0000x reference)
import jax
import jax.numpy as jnp
from jax.experimental import pallas as pl


def kernel(test_feature, train_features, train_labels):
    raise NotImplementedError("write your pallas kernel here")


# speedup vs baseline: 6.3416x; 6.3416x over previous
import functools

import jax
import jax.numpy as jnp
from jax import lax
from jax.experimental import pallas as pl
from jax.experimental.pallas import tpu as pltpu
from jax.experimental.pallas import tpu_sc as plsc

BS = 2
NP = 256
D = 384
N = 20000
P = 14
IMG = 224
K = 20
C = 21

N_PAD = 20480
N_BLK = 2048
N_STEPS = N_PAD // N_BLK
R = BS * NP
DL = 256
NEG = -1e30


def _topk_body(x_ref, a_ref, w_ref, idx_ref, run_v, run_i):
    n = pl.program_id(0)

    @pl.when(n == 0)
    def _init():
        run_v[...] = jnp.full((R, K), NEG, jnp.float32)
        run_i[...] = jnp.full((R, K), N_PAD, jnp.int32)

    sim = jnp.dot(x_ref[...], a_ref[...],
                  preferred_element_type=jnp.float32)
    col = lax.broadcasted_iota(jnp.int32, (R, N_BLK), 1) + n * N_BLK
    sim = jnp.where(col < N, sim, NEG)

    t = jnp.concatenate([run_v[...], sim], axis=1)
    ti = jnp.concatenate([run_i[...], col], axis=1)
    lane = lax.broadcasted_iota(jnp.int32, t.shape, 1)
    big = jnp.int32(2**30)

    vs = []
    ids = []
    for _ in range(K):
        m = jnp.max(t, axis=1, keepdims=True)
        eq = t == m
        pos = jnp.min(jnp.where(eq, lane, big), axis=1, keepdims=True)
        sel = lane == pos
        gi = jnp.min(jnp.where(sel, ti, big), axis=1, keepdims=True)
        vs.append(m)
        ids.append(gi)
        t = jnp.where(sel, NEG, t)
    new_v = jnp.concatenate(vs, axis=1)
    new_i = jnp.concatenate(ids, axis=1)
    run_v[...] = new_v
    run_i[...] = new_i

    @pl.when(n == N_STEPS - 1)
    def _fin():
        m = jnp.max(new_v, axis=1, keepdims=True)
        e = jnp.exp(new_v - m)
        w_ref[...] = e / jnp.sum(e, axis=1, keepdims=True)
        idx_ref[...] = new_i


def _topk_softmax(x, a_pad):
    return pl.pallas_call(
        _topk_body,
        grid=(N_STEPS,),
        in_specs=[
            pl.BlockSpec((R, D), lambda n: (0, 0)),
            pl.BlockSpec((D, N_BLK), lambda n: (0, n)),
        ],
        out_specs=[
            pl.BlockSpec((R, K), lambda n: (0, 0)),
            pl.BlockSpec((R, K), lambda n: (0, 0)),
        ],
        out_shape=[
            jax.ShapeDtypeStruct((R, K), jnp.float32),
            jax.ShapeDtypeStruct((R, K), jnp.int32),
        ],
        scratch_shapes=[
            pltpu.VMEM((R, K), jnp.float32),
            pltpu.VMEM((R, K), jnp.int32),
        ],
    )(x, a_pad)


def _sc_gather(table, idx_flat):
    info = plsc.get_sparse_core_info()
    nw = info.num_cores * info.num_subcores
    b = idx_flat.shape[0]
    b_per_w = b // nw
    mesh = plsc.VectorSubcoreMesh(core_axis_name="c", subcore_axis_name="s")

    @functools.partial(
        pl.kernel, mesh=mesh,
        out_type=jax.ShapeDtypeStruct((b, DL), jnp.int32),
        scratch_types=[
            pltpu.VMEM((b_per_w,), jnp.int32),
            pltpu.VMEM((b_per_w, DL), jnp.int32),
            pltpu.SemaphoreType.DMA,
        ],
    )
    def gather_k(table_hbm, idx_hbm, out_hbm, idx_v, rows_v, sem):
        wid = lax.axis_index("s") * info.num_cores + lax.axis_index("c")
        base = wid * b_per_w
        pltpu.sync_copy(idx_hbm.at[pl.ds(base, b_per_w)], idx_v)
        pltpu.async_copy(table_hbm.at[idx_v], rows_v, sem).wait()
        pltpu.sync_copy(rows_v, out_hbm.at[pl.ds(base, b_per_w)])

    return gather_k(table, idx_flat)


def _vote_body(w_ref, g_ref, o_ref):
    w = w_ref[...][:, :, None]
    g = g_ref[...]
    best_v = jnp.sum(jnp.where(g == 0, w, 0.0), axis=1)
    best_c = jnp.zeros(best_v.shape, jnp.int32)
    for c in range(1, C):
        v = jnp.sum(jnp.where(g == c, w, 0.0), axis=1)
        upd = v > best_v
        best_v = jnp.where(upd, v, best_v)
        best_c = jnp.where(upd, c, best_c)
    o_ref[...] = best_c


def _vote(w, g):
    rb = 64
    return pl.pallas_call(
        _vote_body,
        grid=(R // rb,),
        in_specs=[
            pl.BlockSpec((rb, K), lambda i: (i, 0)),
            pl.BlockSpec((rb, K, DL), lambda i: (i, 0, 0)),
        ],
        out_specs=pl.BlockSpec((rb, DL), lambda i: (i, 0)),
        out_shape=jax.ShapeDtypeStruct((R, DL), jnp.int32),
    )(w, g)


@jax.jit
def kernel(test_feature, train_features, train_labels):
    x = test_feature.reshape(R, D)
    a_pad = jnp.pad(train_features, ((0, 0), (0, N_PAD - N)))
    w, idx = _topk_softmax(x, a_pad)

    table = jnp.pad(train_labels.T, ((0, 0), (0, DL - P * P)))
    g = _sc_gather(table, idx.reshape(R * K))

    pred = _vote(w, g.reshape(R, K, DL))[:, :P * P]

    nr = IMG // P
    img = pred.reshape(BS, nr, nr, P, P)
    img = jnp.transpose(img, (0, 1, 3, 2, 4)).reshape(BS, IMG, IMG)
    return img


# speedup vs baseline: 9.9931x; 1.5758x over previous
import functools

import jax
import jax.numpy as jnp
from jax import lax
from jax.experimental import pallas as pl
from jax.experimental.pallas import tpu as pltpu
from jax.experimental.pallas import tpu_sc as plsc

BS = 2
NP = 256
D = 384
N = 20000
P = 14
IMG = 224
K = 20
C = 21

N_PAD = 20480
N_BLK = 2048
N_STEPS = N_PAD // N_BLK
R = BS * NP
DL = 256
NEG = -1e30


def _extract_topk(s, lane, off):
    big = jnp.int32(2**30)
    vs, ids = [], []
    for _ in range(K):
        m = jnp.max(s, axis=1, keepdims=True)
        pos = jnp.min(jnp.where(s == m, lane, big), axis=1, keepdims=True)
        s = jnp.where(lane == pos, NEG, s)
        vs.append(m)
        ids.append(pos + off)
    return jnp.concatenate(vs, axis=1), jnp.concatenate(ids, axis=1)


def _topk_body(x_ref, a_ref, w_ref, idx_ref, run_v, run_i):
    n = pl.program_id(0)

    sim = jnp.dot(x_ref[...], a_ref[...],
                  preferred_element_type=jnp.float32)
    lane = lax.broadcasted_iota(jnp.int32, (R, N_BLK), 1)
    sim = jnp.where(lane + n * N_BLK < N, sim, NEG)

    lv, li = _extract_topk(sim, lane, n * N_BLK)

    @pl.when(n == 0)
    def _first():
        run_v[...] = lv
        run_i[...] = li

    @pl.when(n > 0)
    def _merge():
        t = jnp.concatenate([run_v[...], lv], axis=1)
        ti = jnp.concatenate([run_i[...], li], axis=1)
        lane2 = lax.broadcasted_iota(jnp.int32, t.shape, 1)
        big = jnp.int32(2**30)
        vs, ids = [], []
        for _ in range(K):
            m = jnp.max(t, axis=1, keepdims=True)
            pos = jnp.min(jnp.where(t == m, lane2, big), axis=1,
                          keepdims=True)
            sel = lane2 == pos
            gi = jnp.min(jnp.where(sel, ti, big), axis=1, keepdims=True)
            t = jnp.where(sel, NEG, t)
            vs.append(m)
            ids.append(gi)
        run_v[...] = jnp.concatenate(vs, axis=1)
        run_i[...] = jnp.concatenate(ids, axis=1)

    @pl.when(n == N_STEPS - 1)
    def _fin():
        fv = run_v[...]
        m = jnp.max(fv, axis=1, keepdims=True)
        e = jnp.exp(fv - m)
        w_ref[...] = e / jnp.sum(e, axis=1, keepdims=True)
        idx_ref[...] = run_i[...]


def _topk_softmax(x, a):
    return pl.pallas_call(
        _topk_body,
        grid=(N_STEPS,),
        in_specs=[
            pl.BlockSpec((R, D), lambda n: (0, 0)),
            pl.BlockSpec((D, N_BLK), lambda n: (0, n)),
        ],
        out_specs=[
            pl.BlockSpec((R, K), lambda n: (0, 0)),
            pl.BlockSpec((R, K), lambda n: (0, 0)),
        ],
        out_shape=[
            jax.ShapeDtypeStruct((R, K), jnp.float32),
            jax.ShapeDtypeStruct((R, K), jnp.int32),
        ],
        scratch_shapes=[
            pltpu.VMEM((R, K), jnp.float32),
            pltpu.VMEM((R, K), jnp.int32),
        ],
    )(x, a)


def _sc_gather(table, idx_flat):
    info = plsc.get_sparse_core_info()
    nw = info.num_cores * info.num_subcores
    b = idx_flat.shape[0]
    b_per_w = b // nw
    mesh = plsc.VectorSubcoreMesh(core_axis_name="c", subcore_axis_name="s")

    @functools.partial(
        pl.kernel, mesh=mesh,
        out_type=jax.ShapeDtypeStruct((b, DL), jnp.int32),
        scratch_types=[
            pltpu.VMEM((b_per_w,), jnp.int32),
            pltpu.VMEM((b_per_w, DL), jnp.int32),
            pltpu.SemaphoreType.DMA,
        ],
    )
    def gather_k(table_hbm, idx_hbm, out_hbm, idx_v, rows_v, sem):
        wid = lax.axis_index("s") * info.num_cores + lax.axis_index("c")
        base = wid * b_per_w
        pltpu.sync_copy(idx_hbm.at[pl.ds(base, b_per_w)], idx_v)
        pltpu.async_copy(table_hbm.at[idx_v], rows_v, sem).wait()
        pltpu.sync_copy(rows_v, out_hbm.at[pl.ds(base, b_per_w)])

    return gather_k(table, idx_flat)


def _vote_body(w_ref, g_ref, o_ref):
    w = w_ref[...][:, :, None]
    g = g_ref[...]
    best_v = jnp.sum(jnp.where(g == 0, w, 0.0), axis=1)
    best_c = jnp.zeros(best_v.shape, jnp.int32)
    for c in range(1, C):
        v = jnp.sum(jnp.where(g == c, w, 0.0), axis=1)
        upd = v > best_v
        best_v = jnp.where(upd, v, best_v)
        best_c = jnp.where(upd, c, best_c)
    o_ref[...] = best_c


def _vote(w, g):
    rb = 64
    return pl.pallas_call(
        _vote_body,
        grid=(R // rb,),
        in_specs=[
            pl.BlockSpec((rb, K), lambda i: (i, 0)),
            pl.BlockSpec((rb, K, DL), lambda i: (i, 0, 0)),
        ],
        out_specs=pl.BlockSpec((rb, DL), lambda i: (i, 0)),
        out_shape=jax.ShapeDtypeStruct((R, DL), jnp.int32),
    )(w, g)


@jax.jit
def kernel(test_feature, train_features, train_labels):
    x = test_feature.reshape(R, D)
    w, idx = _topk_softmax(x, train_features)

    table = jnp.pad(train_labels.T, ((0, 0), (0, DL - P * P)))
    g = _sc_gather(table, idx.reshape(R * K))

    pred = _vote(w, g.reshape(R, K, DL))[:, :P * P]

    nr = IMG // P
    img = pred.reshape(BS, nr, nr, P, P)
    img = jnp.transpose(img, (0, 1, 3, 2, 4)).reshape(BS, IMG, IMG)
    return img


# speedup vs baseline: 10.6189x; 1.0626x over previous
import functools

import jax
import jax.numpy as jnp
from jax import lax
from jax.experimental import pallas as pl
from jax.experimental.pallas import tpu as pltpu
from jax.experimental.pallas import tpu_sc as plsc

BS = 2
NP = 256
D = 384
N = 20000
P = 14
IMG = 224
K = 20
C = 21

N_PAD = 20480
N_BLK = 2048
N_STEPS = N_PAD // N_BLK
R = BS * NP
DL = 256
NEG = -1e30


NCH = 8
CW = N_BLK // NCH
COUNTS = (20, 10, 7, 5, 4, 4, 3, 3)
NET8 = ((0, 1), (2, 3), (4, 5), (6, 7), (0, 2), (1, 3), (4, 6), (5, 7),
        (1, 2), (5, 6), (0, 4), (3, 7), (1, 5), (2, 6), (1, 4), (3, 6),
        (2, 4), (3, 5), (3, 4))
BIG = 2**30


def _topk_body(x_ref, a_ref, w_ref, idx_ref, run_v, run_i):
    n = pl.program_id(0)

    sim = jnp.dot(x_ref[...], a_ref[...],
                  preferred_element_type=jnp.float32)
    lane = lax.broadcasted_iota(jnp.int32, (R, N_BLK), 1)
    sim = jnp.where(lane + n * N_BLK < N, sim, NEG)

    lane_c = lax.broadcasted_iota(jnp.int32, (R, CW), 1)
    vals = [sim[:, j * CW:(j + 1) * CW] for j in range(NCH)]
    idxs = [lane_c + j * CW for j in range(NCH)]
    for a, b in NET8:
        va, vb = vals[a], vals[b]
        ia, ib = idxs[a], idxs[b]
        swap = (vb > va) | ((vb == va) & (ib < ia))
        vals[a] = jnp.where(swap, vb, va)
        vals[b] = jnp.where(swap, va, vb)
        idxs[a] = jnp.where(swap, ib, ia)
        idxs[b] = jnp.where(swap, ia, ib)

    off = n * N_BLK
    cvs, cis = [], []
    for j in range(NCH):
        v, ix = vals[j], idxs[j]
        for _ in range(COUNTS[j]):
            m = jnp.max(v, axis=1, keepdims=True)
            gi = jnp.min(jnp.where(v == m, ix, BIG), axis=1, keepdims=True)
            v = jnp.where((v == m) & (ix == gi), NEG, v)
            cvs.append(m)
            cis.append(gi + off)

    @pl.when(n == 0)
    def _init():
        run_v[...] = jnp.full((R, K), NEG, jnp.float32)
        run_i[...] = jnp.full((R, K), 2**30, jnp.int32)

    t = jnp.concatenate([run_v[...]] + cvs, axis=1)
    ti = jnp.concatenate([run_i[...]] + cis, axis=1)
    vs, ids = [], []
    for _ in range(K):
        m = jnp.max(t, axis=1, keepdims=True)
        gi = jnp.min(jnp.where(t == m, ti, BIG), axis=1, keepdims=True)
        t = jnp.where((t == m) & (ti == gi), NEG, t)
        vs.append(m)
        ids.append(gi)
    new_v = jnp.concatenate(vs, axis=1)
    new_i = jnp.concatenate(ids, axis=1)
    run_v[...] = new_v
    run_i[...] = new_i

    @pl.when(n == N_STEPS - 1)
    def _fin():
        m = jnp.max(new_v, axis=1, keepdims=True)
        e = jnp.exp(new_v - m)
        w_ref[...] = e / jnp.sum(e, axis=1, keepdims=True)
        idx_ref[...] = new_i


def _topk_softmax(x, a):
    return pl.pallas_call(
        _topk_body,
        grid=(N_STEPS,),
        in_specs=[
            pl.BlockSpec((R, D), lambda n: (0, 0)),
            pl.BlockSpec((D, N_BLK), lambda n: (0, n)),
        ],
        out_specs=[
            pl.BlockSpec((R, K), lambda n: (0, 0)),
            pl.BlockSpec((R, K), lambda n: (0, 0)),
        ],
        out_shape=[
            jax.ShapeDtypeStruct((R, K), jnp.float32),
            jax.ShapeDtypeStruct((R, K), jnp.int32),
        ],
        scratch_shapes=[
            pltpu.VMEM((R, K), jnp.float32),
            pltpu.VMEM((R, K), jnp.int32),
        ],
    )(x, a)


def _sc_gather(table, idx_flat):
    info = plsc.get_sparse_core_info()
    nw = info.num_cores * info.num_subcores
    b = idx_flat.shape[0]
    b_per_w = b // nw
    mesh = plsc.VectorSubcoreMesh(core_axis_name="c", subcore_axis_name="s")

    @functools.partial(
        pl.kernel, mesh=mesh,
        out_type=jax.ShapeDtypeStruct((b, DL), jnp.int32),
        scratch_types=[
            pltpu.VMEM((b_per_w,), jnp.int32),
            pltpu.VMEM((b_per_w, DL), jnp.int32),
            pltpu.SemaphoreType.DMA,
        ],
    )
    def gather_k(table_hbm, idx_hbm, out_hbm, idx_v, rows_v, sem):
        wid = lax.axis_index("s") * info.num_cores + lax.axis_index("c")
        base = wid * b_per_w
        pltpu.sync_copy(idx_hbm.at[pl.ds(base, b_per_w)], idx_v)
        pltpu.async_copy(table_hbm.at[idx_v], rows_v, sem).wait()
        pltpu.sync_copy(rows_v, out_hbm.at[pl.ds(base, b_per_w)])

    return gather_k(table, idx_flat)


DV = DL


def _vote_body(w_ref, g_ref, o_ref):
    w = w_ref[...][:, :, None]
    g = g_ref[...]
    best_v = jnp.sum(jnp.where(g == 0, w, 0.0), axis=1)
    best_c = jnp.zeros(best_v.shape, jnp.int32)
    for c in range(1, C):
        v = jnp.sum(jnp.where(g == c, w, 0.0), axis=1)
        upd = v > best_v
        best_v = jnp.where(upd, v, best_v)
        best_c = jnp.where(upd, c, best_c)
    o_ref[...] = best_c


def _vote(w, g):
    rb = 64
    return pl.pallas_call(
        _vote_body,
        grid=(R // rb,),
        in_specs=[
            pl.BlockSpec((rb, K), lambda i: (i, 0)),
            pl.BlockSpec((rb, K, DV), lambda i: (i, 0, 0)),
        ],
        out_specs=pl.BlockSpec((rb, DV), lambda i: (i, 0)),
        out_shape=jax.ShapeDtypeStruct((R, DV), jnp.int32),
    )(w, g)


@jax.jit
def kernel(test_feature, train_features, train_labels):
    x = test_feature.reshape(R, D)
    w, idx = _topk_softmax(x, train_features)

    table = jnp.pad(train_labels.T, ((0, 0), (0, DL - P * P)))
    g = _sc_gather(table, idx.reshape(R * K))

    pred = _vote(w, g.reshape(R, K, DL))[:, :P * P]

    nr = IMG // P
    img = pred.reshape(BS, nr, nr, P, P)
    img = jnp.transpose(img, (0, 1, 3, 2, 4)).reshape(BS, IMG, IMG)
    return img


# speedup vs baseline: 10.7801x; 1.0152x over previous
import functools

import jax
import jax.numpy as jnp
from jax import lax
from jax.experimental import pallas as pl
from jax.experimental.pallas import tpu as pltpu
from jax.experimental.pallas import tpu_sc as plsc

BS = 2
NP = 256
D = 384
N = 20000
P = 14
IMG = 224
K = 20
C = 21

N_PAD = 20480
N_BLK = 2048
N_STEPS = N_PAD // N_BLK
R = BS * NP
DL = 256
NEG = -1e30


NCH = 8
CW = N_BLK // NCH
COUNTS = (20, 10, 7, 5, 4, 4, 3, 3)
NET8 = ((0, 1), (2, 3), (4, 5), (6, 7), (0, 2), (1, 3), (4, 6), (5, 7),
        (1, 2), (5, 6), (0, 4), (3, 7), (1, 5), (2, 6), (1, 4), (3, 6),
        (2, 4), (3, 5), (3, 4))
BIG = 2**30


def _topk_body(x_ref, a_ref, l_ref, w_ref, idx_ref, tab_ref, run_v, run_i):
    n = pl.program_id(0)

    lt = jnp.transpose(l_ref[...], (1, 0))
    tab_ref[...] = jnp.pad(lt, ((0, 0), (0, DL - P * P)))

    sim = jnp.dot(x_ref[...], a_ref[...],
                  preferred_element_type=jnp.float32)
    lane = lax.broadcasted_iota(jnp.int32, (R, N_BLK), 1)
    sim = jnp.where(lane + n * N_BLK < N, sim, NEG)

    lane_c = lax.broadcasted_iota(jnp.int32, (R, CW), 1)
    vals = [sim[:, j * CW:(j + 1) * CW] for j in range(NCH)]
    idxs = [lane_c + j * CW for j in range(NCH)]
    for a, b in NET8:
        va, vb = vals[a], vals[b]
        ia, ib = idxs[a], idxs[b]
        swap = (vb > va) | ((vb == va) & (ib < ia))
        vals[a] = jnp.where(swap, vb, va)
        vals[b] = jnp.where(swap, va, vb)
        idxs[a] = jnp.where(swap, ib, ia)
        idxs[b] = jnp.where(swap, ia, ib)

    off = n * N_BLK
    cvs, cis = [], []
    for j in range(NCH):
        v, ix = vals[j], idxs[j]
        for _ in range(COUNTS[j]):
            m = jnp.max(v, axis=1, keepdims=True)
            gi = jnp.min(jnp.where(v == m, ix, BIG), axis=1, keepdims=True)
            v = jnp.where((v == m) & (ix == gi), NEG, v)
            cvs.append(m)
            cis.append(gi + off)

    @pl.when(n == 0)
    def _init():
        run_v[...] = jnp.full((R, K), NEG, jnp.float32)
        run_i[...] = jnp.full((R, K), 2**30, jnp.int32)

    t = jnp.concatenate([run_v[...]] + cvs, axis=1)
    ti = jnp.concatenate([run_i[...]] + cis, axis=1)
    vs, ids = [], []
    for _ in range(K):
        m = jnp.max(t, axis=1, keepdims=True)
        gi = jnp.min(jnp.where(t == m, ti, BIG), axis=1, keepdims=True)
        t = jnp.where((t == m) & (ti == gi), NEG, t)
        vs.append(m)
        ids.append(gi)
    new_v = jnp.concatenate(vs, axis=1)
    new_i = jnp.concatenate(ids, axis=1)
    run_v[...] = new_v
    run_i[...] = new_i

    @pl.when(n == N_STEPS - 1)
    def _fin():
        m = jnp.max(new_v, axis=1, keepdims=True)
        e = jnp.exp(new_v - m)
        w_ref[...] = e / jnp.sum(e, axis=1, keepdims=True)
        idx_ref[...] = new_i


def _topk_softmax(x, a, labels):
    return pl.pallas_call(
        _topk_body,
        grid=(N_STEPS,),
        in_specs=[
            pl.BlockSpec((R, D), lambda n: (0, 0)),
            pl.BlockSpec((D, N_BLK), lambda n: (0, n)),
            pl.BlockSpec((P * P, N_BLK), lambda n: (0, n)),
        ],
        out_specs=[
            pl.BlockSpec((R, K), lambda n: (0, 0)),
            pl.BlockSpec((R, K), lambda n: (0, 0)),
            pl.BlockSpec((N_BLK, DL), lambda n: (n, 0)),
        ],
        out_shape=[
            jax.ShapeDtypeStruct((R, K), jnp.float32),
            jax.ShapeDtypeStruct((R, K), jnp.int32),
            jax.ShapeDtypeStruct((N_PAD, DL), jnp.int32),
        ],
        scratch_shapes=[
            pltpu.VMEM((R, K), jnp.float32),
            pltpu.VMEM((R, K), jnp.int32),
        ],
    )(x, a, labels)


def _sc_gather(table, idx_flat):
    info = plsc.get_sparse_core_info()
    nw = info.num_cores * info.num_subcores
    b = idx_flat.shape[0]
    b_per_w = b // nw
    mesh = plsc.VectorSubcoreMesh(core_axis_name="c", subcore_axis_name="s")

    @functools.partial(
        pl.kernel, mesh=mesh,
        out_type=jax.ShapeDtypeStruct((b, DL), jnp.int32),
        scratch_types=[
            pltpu.VMEM((b_per_w,), jnp.int32),
            pltpu.VMEM((b_per_w, DL), jnp.int32),
            pltpu.SemaphoreType.DMA,
        ],
    )
    def gather_k(table_hbm, idx_hbm, out_hbm, idx_v, rows_v, sem):
        wid = lax.axis_index("s") * info.num_cores + lax.axis_index("c")
        base = wid * b_per_w
        pltpu.sync_copy(idx_hbm.at[pl.ds(base, b_per_w)], idx_v)
        pltpu.async_copy(table_hbm.at[idx_v], rows_v, sem).wait()
        pltpu.sync_copy(rows_v, out_hbm.at[pl.ds(base, b_per_w)])

    return gather_k(table, idx_flat)


DV = DL


def _vote_body(w_ref, g_ref, o_ref):
    w = w_ref[...][:, :, None]
    g = g_ref[...]
    best_v = jnp.sum(jnp.where(g == 0, w, 0.0), axis=1)
    best_c = jnp.zeros(best_v.shape, jnp.int32)
    for c in range(1, C):
        v = jnp.sum(jnp.where(g == c, w, 0.0), axis=1)
        upd = v > best_v
        best_v = jnp.where(upd, v, best_v)
        best_c = jnp.where(upd, c, best_c)
    o_ref[...] = best_c


def _vote(w, g):
    rb = 64
    return pl.pallas_call(
        _vote_body,
        grid=(R // rb,),
        in_specs=[
            pl.BlockSpec((rb, K), lambda i: (i, 0)),
            pl.BlockSpec((rb, K, DV), lambda i: (i, 0, 0)),
        ],
        out_specs=pl.BlockSpec((rb, DV), lambda i: (i, 0)),
        out_shape=jax.ShapeDtypeStruct((R, DV), jnp.int32),
    )(w, g)


@jax.jit
def kernel(test_feature, train_features, train_labels):
    x = test_feature.reshape(R, D)
    w, idx, table = _topk_softmax(x, train_features, train_labels)
    g = _sc_gather(table, idx.reshape(R * K))

    pred = _vote(w, g.reshape(R, K, DL))[:, :P * P]

    nr = IMG // P
    img = pred.reshape(BS, nr, nr, P, P)
    img = jnp.transpose(img, (0, 1, 3, 2, 4)).reshape(BS, IMG, IMG)
    return img
